# baseline (speedup 1.0000x reference)
import jax
import jax.numpy as jnp
from jax.experimental import pallas as pl
from jax.experimental.pallas import tpu as pltpu

_B, _T, _H, _W, _D = 8, 8, 14, 14, 768
_HW = _H * _W
_L = _T * _HW
_EPS = 1e-06


def _fused_kernel(x_ref, te_ref, ve_ref, he_ref, g_ref, b_ref, o_ref,
                  vh_ref, pos_ref):
    b = pl.program_id(0)
    t = pl.program_id(1)

    @pl.when((b == 0) & (t == 0))
    def _build_vh():
        row = jax.lax.broadcasted_iota(jnp.int32, (_HW, _H), 0)
        col = jax.lax.broadcasted_iota(jnp.int32, (_HW, _H), 1)
        v_oh = (row // _W == col).astype(jnp.float32)
        h_oh = (row % _W == col).astype(jnp.float32)
        vh_ref[:] = (
            jax.lax.dot(v_oh, ve_ref[:], preferred_element_type=jnp.float32)
            + jax.lax.dot(h_oh, he_ref[:], preferred_element_type=jnp.float32)
        )

    @pl.when(b == 0)
    def _build_pos():
        pos = vh_ref[:] + te_ref[pl.ds(t, 1), :]
        mean = jnp.mean(pos, axis=-1, keepdims=True)
        c = pos - mean
        var = jnp.mean(c * c, axis=-1, keepdims=True)
        ln = c * jax.lax.rsqrt(var + _EPS) * g_ref[:] + b_ref[:]
        pos_ref[pl.ds(t, 1)] = ln[None]

    o_ref[0, 0] = x_ref[0, 0] + pos_ref[pl.ds(t, 1)][0]


def kernel(inputs, temporal_emb, vertical_emb, horizontal_emb, gamma, beta,
           dimensions):
    x = inputs.reshape(_B, _T, _HW, _D)
    g = gamma.reshape(1, _D)
    be = beta.reshape(1, _D)
    out = pl.pallas_call(
        _fused_kernel,
        grid=(_B, _T),
        in_specs=[
            pl.BlockSpec((1, 1, _HW, _D), lambda b, t: (b, t, 0, 0)),
            pl.BlockSpec((_T, _D), lambda b, t: (0, 0)),
            pl.BlockSpec((_H, _D), lambda b, t: (0, 0)),
            pl.BlockSpec((_W, _D), lambda b, t: (0, 0)),
            pl.BlockSpec((1, _D), lambda b, t: (0, 0)),
            pl.BlockSpec((1, _D), lambda b, t: (0, 0)),
        ],
        out_specs=pl.BlockSpec((1, 1, _HW, _D), lambda b, t: (b, t, 0, 0)),
        out_shape=jax.ShapeDtypeStruct((_B, _T, _HW, _D), jnp.float32),
        scratch_shapes=[
            pltpu.VMEM((_HW, _D), jnp.float32),
            pltpu.VMEM((_T, _HW, _D), jnp.float32),
        ],
        compiler_params=pltpu.CompilerParams(
            dimension_semantics=("arbitrary", "arbitrary"),
        ),
    )(x, temporal_emb, vertical_emb, horizontal_emb, g, be)
    return out.reshape(_B, _L, _D)


# speedup vs baseline: 4.8547x; 4.8547x over previous
import jax
import jax.numpy as jnp
from jax.experimental import pallas as pl
from jax.experimental.pallas import tpu as pltpu

_B, _T, _H, _W, _D = 8, 8, 14, 14, 768
_HW = _H * _W
_L = _T * _HW
_EPS = 1e-06
_BL = 392
_NJ = _L // _BL


def _fused_kernel(x_ref, te_ref, ve_ref, he_ref, g_ref, b_ref, o_ref,
                  pos_ref):
    b = pl.program_id(0)
    j = pl.program_id(1)

    @pl.when((b == 0) & (j == 0))
    def _build_pos():
        def onehot(idx_fn, n):
            row = jax.lax.broadcasted_iota(jnp.int32, (_L, n), 0)
            col = jax.lax.broadcasted_iota(jnp.int32, (_L, n), 1)
            return (idx_fn(row) == col).astype(jnp.float32)

        pos = (
            jax.lax.dot(onehot(lambda r: r // _HW, _T), te_ref[:],
                        preferred_element_type=jnp.float32)
            + jax.lax.dot(onehot(lambda r: (r // _W) % _H, _H), ve_ref[:],
                          preferred_element_type=jnp.float32)
            + jax.lax.dot(onehot(lambda r: r % _W, _W), he_ref[:],
                          preferred_element_type=jnp.float32)
        )
        mean = jnp.mean(pos, axis=-1, keepdims=True)
        c = pos - mean
        var = jnp.mean(c * c, axis=-1, keepdims=True)
        pos_ref[:] = c * jax.lax.rsqrt(var + _EPS) * g_ref[:] + b_ref[:]

    o_ref[0] = x_ref[0] + pos_ref[pl.ds(j * _BL, _BL), :]


def kernel(inputs, temporal_emb, vertical_emb, horizontal_emb, gamma, beta,
           dimensions):
    g = gamma.reshape(1, _D)
    be = beta.reshape(1, _D)
    out = pl.pallas_call(
        _fused_kernel,
        grid=(_B, _NJ),
        in_specs=[
            pl.BlockSpec((1, _BL, _D), lambda b, j: (b, j, 0)),
            pl.BlockSpec((_T, _D), lambda b, j: (0, 0)),
            pl.BlockSpec((_H, _D), lambda b, j: (0, 0)),
            pl.BlockSpec((_W, _D), lambda b, j: (0, 0)),
            pl.BlockSpec((1, _D), lambda b, j: (0, 0)),
            pl.BlockSpec((1, _D), lambda b, j: (0, 0)),
        ],
        out_specs=pl.BlockSpec((1, _BL, _D), lambda b, j: (b, j, 0)),
        out_shape=jax.ShapeDtypeStruct((_B, _L, _D), jnp.float32),
        scratch_shapes=[
            pltpu.VMEM((_L, _D), jnp.float32),
        ],
        compiler_params=pltpu.CompilerParams(
            dimension_semantics=("arbitrary", "arbitrary"),
        ),
    )(inputs, temporal_emb, vertical_emb, horizontal_emb, g, be)
    return out


# speedup vs baseline: 6.0880x; 1.2540x over previous
import jax
import jax.numpy as jnp
from jax.experimental import pallas as pl
from jax.experimental.pallas import tpu as pltpu

_B, _T, _H, _W, _D = 8, 8, 14, 14, 768
_HW = _H * _W
_L = _T * _HW
_EPS = 1e-06
_BL = 784
_NJ = _L // _BL


def _fused_kernel(x_ref, te_ref, ve_ref, he_ref, g_ref, b_ref, o_ref,
                  pos_ref):
    b = pl.program_id(0)
    j = pl.program_id(1)

    @pl.when((b == 0) & (j == 0))
    def _build_pos():
        def onehot(idx_fn, n):
            row = jax.lax.broadcasted_iota(jnp.int32, (_L, n), 0)
            col = jax.lax.broadcasted_iota(jnp.int32, (_L, n), 1)
            return (idx_fn(row) == col).astype(jnp.float32)

        pos = (
            jax.lax.dot(onehot(lambda r: r // _HW, _T), te_ref[:],
                        preferred_element_type=jnp.float32)
            + jax.lax.dot(onehot(lambda r: (r // _W) % _H, _H), ve_ref[:],
                          preferred_element_type=jnp.float32)
            + jax.lax.dot(onehot(lambda r: r % _W, _W), he_ref[:],
                          preferred_element_type=jnp.float32)
        )
        mean = jnp.mean(pos, axis=-1, keepdims=True)
        c = pos - mean
        var = jnp.mean(c * c, axis=-1, keepdims=True)
        pos_ref[:] = c * jax.lax.rsqrt(var + _EPS) * g_ref[:] + b_ref[:]

    o_ref[0] = x_ref[0] + pos_ref[pl.ds(j * _BL, _BL), :]


def kernel(inputs, temporal_emb, vertical_emb, horizontal_emb, gamma, beta,
           dimensions):
    g = gamma.reshape(1, _D)
    be = beta.reshape(1, _D)
    out = pl.pallas_call(
        _fused_kernel,
        grid=(_B, _NJ),
        in_specs=[
            pl.BlockSpec((1, _BL, _D), lambda b, j: (b, j, 0)),
            pl.BlockSpec((_T, _D), lambda b, j: (0, 0)),
            pl.BlockSpec((_H, _D), lambda b, j: (0, 0)),
            pl.BlockSpec((_W, _D), lambda b, j: (0, 0)),
            pl.BlockSpec((1, _D), lambda b, j: (0, 0)),
            pl.BlockSpec((1, _D), lambda b, j: (0, 0)),
        ],
        out_specs=pl.BlockSpec((1, _BL, _D), lambda b, j: (b, j, 0)),
        out_shape=jax.ShapeDtypeStruct((_B, _L, _D), jnp.float32),
        scratch_shapes=[
            pltpu.VMEM((_L, _D), jnp.float32),
        ],
        compiler_params=pltpu.CompilerParams(
            dimension_semantics=("arbitrary", "arbitrary"),
        ),
    )(inputs, temporal_emb, vertical_emb, horizontal_emb, g, be)
    return out


# speedup vs baseline: 6.4365x; 1.0572x over previous
import jax
import jax.numpy as jnp
from jax.experimental import pallas as pl
from jax.experimental.pallas import tpu as pltpu

_B, _T, _H, _W, _D = 8, 8, 14, 14, 768
_HW = _H * _W
_L = _T * _HW
_EPS = 1e-06
_BL = 1568
_NJ = _L // _BL


def _fused_kernel(x_ref, te_ref, ve_ref, he_ref, g_ref, b_ref, o_ref,
                  pos_ref):
    b = pl.program_id(0)
    j = pl.program_id(1)

    @pl.when((b == 0) & (j == 0))
    def _build_pos():
        def onehot(idx_fn, n):
            row = jax.lax.broadcasted_iota(jnp.int32, (_L, n), 0)
            col = jax.lax.broadcasted_iota(jnp.int32, (_L, n), 1)
            return (idx_fn(row) == col).astype(jnp.float32)

        pos = (
            jax.lax.dot(onehot(lambda r: r // _HW, _T), te_ref[:],
                        preferred_element_type=jnp.float32)
            + jax.lax.dot(onehot(lambda r: (r // _W) % _H, _H), ve_ref[:],
                          preferred_element_type=jnp.float32)
            + jax.lax.dot(onehot(lambda r: r % _W, _W), he_ref[:],
                          preferred_element_type=jnp.float32)
        )
        mean = jnp.mean(pos, axis=-1, keepdims=True)
        c = pos - mean
        var = jnp.mean(c * c, axis=-1, keepdims=True)
        pos_ref[:] = c * jax.lax.rsqrt(var + _EPS) * g_ref[:] + b_ref[:]

    o_ref[0] = x_ref[0] + pos_ref[pl.ds(j * _BL, _BL), :]


def kernel(inputs, temporal_emb, vertical_emb, horizontal_emb, gamma, beta,
           dimensions):
    g = gamma.reshape(1, _D)
    be = beta.reshape(1, _D)
    out = pl.pallas_call(
        _fused_kernel,
        grid=(_B, _NJ),
        in_specs=[
            pl.BlockSpec((1, _BL, _D), lambda b, j: (b, j, 0)),
            pl.BlockSpec((_T, _D), lambda b, j: (0, 0)),
            pl.BlockSpec((_H, _D), lambda b, j: (0, 0)),
            pl.BlockSpec((_W, _D), lambda b, j: (0, 0)),
            pl.BlockSpec((1, _D), lambda b, j: (0, 0)),
            pl.BlockSpec((1, _D), lambda b, j: (0, 0)),
        ],
        out_specs=pl.BlockSpec((1, _BL, _D), lambda b, j: (b, j, 0)),
        out_shape=jax.ShapeDtypeStruct((_B, _L, _D), jnp.float32),
        scratch_shapes=[
            pltpu.VMEM((_L, _D), jnp.float32),
        ],
        compiler_params=pltpu.CompilerParams(
            dimension_semantics=("arbitrary", "arbitrary"),
        ),
    )(inputs, temporal_emb, vertical_emb, horizontal_emb, g, be)
    return out


# speedup vs baseline: 6.8868x; 1.0700x over previous
import jax
import jax.numpy as jnp
from jax.experimental import pallas as pl
from jax.experimental.pallas import tpu as pltpu

_B, _T, _H, _W, _D = 8, 8, 14, 14, 768
_HW = _H * _W
_L = _T * _HW
_EPS = 1e-06
_BL = 1568
_NJ = _L // _BL
_BB = 2


def _fused_kernel(x_ref, te_ref, ve_ref, he_ref, g_ref, b_ref, o_ref,
                  pos_ref):
    b = pl.program_id(0)
    j = pl.program_id(1)

    @pl.when((b == 0) & (j == 0))
    def _build_pos():
        def onehot(idx_fn, n):
            row = jax.lax.broadcasted_iota(jnp.int32, (_L, n), 0)
            col = jax.lax.broadcasted_iota(jnp.int32, (_L, n), 1)
            return (idx_fn(row) == col).astype(jnp.float32)

        pos = (
            jax.lax.dot(onehot(lambda r: r // _HW, _T), te_ref[:],
                        preferred_element_type=jnp.float32)
            + jax.lax.dot(onehot(lambda r: (r // _W) % _H, _H), ve_ref[:],
                          preferred_element_type=jnp.float32)
            + jax.lax.dot(onehot(lambda r: r % _W, _W), he_ref[:],
                          preferred_element_type=jnp.float32)
        )
        mean = jnp.mean(pos, axis=-1, keepdims=True)
        c = pos - mean
        var = jnp.mean(c * c, axis=-1, keepdims=True)
        pos_ref[:] = c * jax.lax.rsqrt(var + _EPS) * g_ref[:] + b_ref[:]

    o_ref[:] = x_ref[:] + pos_ref[pl.ds(j * _BL, _BL), :][None]


def kernel(inputs, temporal_emb, vertical_emb, horizontal_emb, gamma, beta,
           dimensions):
    g = gamma.reshape(1, _D)
    be = beta.reshape(1, _D)
    out = pl.pallas_call(
        _fused_kernel,
        grid=(_B // _BB, _NJ),
        in_specs=[
            pl.BlockSpec((_BB, _BL, _D), lambda b, j: (b, j, 0)),
            pl.BlockSpec((_T, _D), lambda b, j: (0, 0)),
            pl.BlockSpec((_H, _D), lambda b, j: (0, 0)),
            pl.BlockSpec((_W, _D), lambda b, j: (0, 0)),
            pl.BlockSpec((1, _D), lambda b, j: (0, 0)),
            pl.BlockSpec((1, _D), lambda b, j: (0, 0)),
        ],
        out_specs=pl.BlockSpec((_BB, _BL, _D), lambda b, j: (b, j, 0)),
        out_shape=jax.ShapeDtypeStruct((_B, _L, _D), jnp.float32),
        scratch_shapes=[
            pltpu.VMEM((_L, _D), jnp.float32),
        ],
        compiler_params=pltpu.CompilerParams(
            dimension_semantics=("arbitrary", "arbitrary"),
        ),
    )(inputs, temporal_emb, vertical_emb, horizontal_emb, g, be)
    return out
